# baseline (speedup 1.0000x reference)
import functools

import jax
import jax.numpy as jnp
from jax import lax
from jax.experimental import pallas as pl
from jax.experimental.pallas import tpu as pltpu
from jax.experimental.pallas import tpu_sc as plsc

_NC = 2
_NS = 16
_L = 16
_NW = _NC * _NS

_N = 10000
_D = 128
_E = 320000
_EPW = _E // _NW
_CH = 80
_NCH = _EPW // _CH
_RPS = _N // _NS
_ZR = 125
_HR = _N // _L

_mesh = plsc.VectorSubcoreMesh(core_axis_name="c", subcore_axis_name="s")
_sc_params = pltpu.CompilerParams(needs_layout_passes=False)


@functools.partial(
    pl.kernel,
    out_type=jax.ShapeDtypeStruct((_NC, _HR, _L), jnp.float32),
    mesh=_mesh,
    scratch_types=[
        pltpu.VMEM((_EPW,), jnp.int32),
        pltpu.VMEM((_HR, _L), jnp.float32),
        pltpu.VMEM((5, _ZR), jnp.int32),
        pltpu.VMEM_SHARED((_HR, _L), jnp.float32),
    ],
    compiler_params=_sc_params,
)
def _sc_degree_hist(dst_hbm, rix_hbm, dep_hbm, out_hbm,
                    idx_v, hist_v, rix_v, hist_sh):
    c = lax.axis_index("c")
    s = lax.axis_index("s")
    wid = s * _NC + c
    pltpu.sync_copy(dst_hbm.at[wid], idx_v)
    pltpu.sync_copy(rix_hbm, rix_v)
    z16 = jnp.zeros((_L,), jnp.float32)

    @pl.loop(0, _HR)
    def _zero(r):
        hist_v[r, :] = z16

    @pl.when(s == 0)
    def _():
        pltpu.sync_copy(hist_v, hist_sh)

    plsc.subcore_barrier()

    ones = jnp.ones((_L,), jnp.float32)
    four = jnp.full((_L,), 4, jnp.int32)
    fifteen = jnp.full((_L,), 15, jnp.int32)

    @pl.loop(0, _EPW, step=_L)
    def _hist(i):
        idx = idx_v[pl.ds(i, _L)]
        row = jnp.right_shift(idx, four)
        colv = jnp.bitwise_and(idx, fifteen)
        plsc.addupdate_scatter(hist_v, [row, colv], ones)

    @pl.loop(0, 5)
    def _combine(k):
        pltpu.sync_copy(hist_v.at[pl.ds(k * _ZR, _ZR)],
                        hist_sh.at[rix_v.at[k]], add=True)

    plsc.subcore_barrier()

    @pl.when(s == 0)
    def _():
        pltpu.sync_copy(hist_sh, out_hbm.at[c])


_HALF = _N // _NC
_TRASH = 64
_AROWS = 5120
_CHP = 80
_BCAP = 5760
_BCH = _BCAP // _CHP
_NSEG = 2 * _BCH


@functools.partial(
    pl.kernel,
    out_type=[
        jax.ShapeDtypeStruct((_NW, 2, _BCH, _CHP), jnp.int32),
        jax.ShapeDtypeStruct((_NW, 2, _BCH, _CHP), jnp.int32),
    ],
    mesh=_mesh,
    scratch_types=[
        pltpu.VMEM((_EPW,), jnp.int32),
        pltpu.VMEM((_EPW,), jnp.int32),
        pltpu.VMEM((_BCH, _CHP), jnp.int32),
        pltpu.VMEM((_BCH, _CHP), jnp.int32),
        pltpu.VMEM((_BCH, _CHP), jnp.int32),
        pltpu.VMEM((_BCH, _CHP), jnp.int32),
    ],
    compiler_params=_sc_params,
)
def _sc_partition(src_hbm, dst_hbm, psrc_hbm, pdst_hbm,
                  sv, dv, cs0, cd0, cs1, cd1):
    c = lax.axis_index("c")
    s = lax.axis_index("s")
    wid = s * _NC + c
    pltpu.sync_copy(src_hbm.at[wid], sv)
    pltpu.sync_copy(dst_hbm.at[wid], dv)
    iota = lax.iota(jnp.int32, _L)
    chw = jnp.full((_L,), _CHP, jnp.int32)
    halfv = jnp.full((_L,), _HALF, jnp.int32)
    capv = jnp.full((_L,), _BCAP - 1, jnp.int32)
    zsrc = jnp.zeros((_L,), jnp.int32)
    trash = halfv + jnp.bitwise_and(iota, jnp.full((_L,), _TRASH - 1,
                                                   jnp.int32))

    @pl.loop(0, _BCH)
    def _prefill(r):
        @pl.loop(0, _CHP, step=_L)
        def _(j):
            cs0[r, pl.ds(j, _L)] = zsrc
            cd0[r, pl.ds(j, _L)] = trash
            cs1[r, pl.ds(j, _L)] = zsrc
            cd1[r, pl.ds(j, _L)] = trash

    def vec_body(v, carry):
        b0, b1 = carry
        s16 = sv[pl.ds(v * _L, _L)]
        d16 = dv[pl.ds(v * _L, _L)]
        m0 = d16 < halfv
        m1 = jnp.logical_not(m0)
        cum0 = plsc.cumsum(m0.astype(jnp.int32))
        cum1 = plsc.cumsum(m1.astype(jnp.int32))
        pos0 = jnp.minimum(cum0 + (b0 - 1), capv)
        pos1 = jnp.minimum(cum1 + (b1 - 1), capv)
        r0 = lax.div(pos0, chw)
        c0 = pos0 - r0 * chw
        r1 = lax.div(pos1, chw)
        c1 = pos1 - r1 * chw
        plsc.store_scatter(cs0, [r0, c0], s16, mask=m0)
        plsc.store_scatter(cd0, [r0, c0], d16, mask=m0)
        plsc.store_scatter(cs1, [r1, c1], s16, mask=m1)
        plsc.store_scatter(cd1, [r1, c1], d16 - halfv, mask=m1)
        return (b0 + jnp.sum(m0.astype(jnp.int32)),
                b1 + jnp.sum(m1.astype(jnp.int32)))

    b0, b1 = lax.fori_loop(0, _EPW // _L, vec_body,
                           (jnp.int32(0), jnp.int32(0)))

    @pl.when(b0 + b1 == _EPW)
    def _writeout():
        pltpu.sync_copy(cs0, psrc_hbm.at[wid, 0])
        pltpu.sync_copy(cd0, pdst_hbm.at[wid, 0])
        pltpu.sync_copy(cs1, psrc_hbm.at[wid, 1])
        pltpu.sync_copy(cd1, pdst_hbm.at[wid, 1])


@functools.partial(
    pl.kernel,
    out_type=jax.ShapeDtypeStruct((_NC, _HALF, _D), jnp.float32),
    mesh=_mesh,
    scratch_types=[
        pltpu.VMEM((_NSEG, _CHP), jnp.int32),
        pltpu.VMEM((_NSEG, _CHP), jnp.int32),
        pltpu.VMEM((_CHP, _D), jnp.float32),
        pltpu.VMEM((_CHP, _D), jnp.float32),
        pltpu.VMEM_SHARED((_AROWS, _D), jnp.float32),
        pltpu.SemaphoreType.DMA,
        pltpu.SemaphoreType.DMA,
    ],
    compiler_params=_sc_params,
)
def _sc_edge_agg(hs_hbm, psrc_hbm, pdst_hbm, out_hbm,
                 src_v, dst_v, rows_a, rows_b, agg_sh, gsem_a, gsem_b):
    c = lax.axis_index("c")
    s = lax.axis_index("s")
    z16 = jnp.zeros((_L,), jnp.float32)

    pltpu.sync_copy(psrc_hbm.at[s * 2, c], src_v.at[pl.ds(0, _BCH)])
    pltpu.sync_copy(pdst_hbm.at[s * 2, c], dst_v.at[pl.ds(0, _BCH)])
    pltpu.sync_copy(psrc_hbm.at[s * 2 + 1, c], src_v.at[pl.ds(_BCH, _BCH)])
    pltpu.sync_copy(pdst_hbm.at[s * 2 + 1, c], dst_v.at[pl.ds(_BCH, _BCH)])

    @pl.loop(0, _CHP)
    def _zero(r):
        @pl.loop(0, _D, step=_L)
        def _(j):
            rows_a[r, pl.ds(j, _L)] = z16

    @pl.loop(0, 320, step=80)
    def _zero_spmem(r0):
        pltpu.sync_copy(rows_a.at[pl.ds(0, 80)],
                        agg_sh.at[pl.ds(s * 320 + r0, 80)])

    pltpu.async_copy(hs_hbm.at[src_v.at[0]], rows_a, gsem_a)
    plsc.subcore_barrier()

    @pl.loop(0, _NSEG, step=2)
    def _edges(j):
        pltpu.async_copy(hs_hbm.at[src_v.at[j + 1]], rows_b, gsem_b)
        pltpu.make_async_copy(hs_hbm.at[src_v.at[j]], rows_a, gsem_a).wait()
        pltpu.sync_copy(rows_a, agg_sh.at[dst_v.at[j]], add=True)

        @pl.when(j + 2 < _NSEG)
        def _():
            pltpu.async_copy(hs_hbm.at[src_v.at[j + 2]], rows_a, gsem_a)

        pltpu.make_async_copy(hs_hbm.at[src_v.at[j + 1]], rows_b, gsem_b).wait()
        pltpu.sync_copy(rows_b, agg_sh.at[dst_v.at[j + 1]], add=True)

    plsc.subcore_barrier()

    @pl.when(s < 5)
    def _writeout():
        pltpu.sync_copy(agg_sh.at[pl.ds(s * 1000, 1000)],
                        out_hbm.at[c, pl.ds(s * 1000, 1000)])


def _mm_body(x_ref, w_ref, o_ref):
    o_ref[...] = jnp.dot(x_ref[...], w_ref[...],
                         preferred_element_type=jnp.float32)


def _scale_body(h_ref, deg_ref, o_ref):
    o_ref[...] = h_ref[...] * lax.rsqrt(deg_ref[...])


def _mid_body(agg_ref, hs_ref, deg_ref, b1_ref, gamma_ref, beta_ref,
              w2_ref, o_ref):
    dinv = lax.rsqrt(deg_ref[...])
    t = (agg_ref[...] + hs_ref[...]) * dinv + b1_ref[...]
    mu = jnp.mean(t, axis=0, keepdims=True)
    var = jnp.mean(jnp.square(t - mu), axis=0, keepdims=True)
    hn = (t - mu) * lax.rsqrt(var + 1e-5) * gamma_ref[...] + beta_ref[...]
    h = jnp.maximum(hn, 0.0)
    o_ref[...] = jnp.dot(h, w2_ref[...],
                         preferred_element_type=jnp.float32) * dinv


def _fin_body(agg_ref, hs_ref, deg_ref, b2_ref, o_ref):
    o_ref[...] = ((agg_ref[...] + hs_ref[...]) * lax.rsqrt(deg_ref[...])
                  + b2_ref[...])


def kernel(x, edge_index, W1, b1, gamma, beta, W2, b2):
    n, d = x.shape
    src2 = edge_index[0].astype(jnp.int32).reshape(_NW, _EPW)
    dst2 = edge_index[1].astype(jnp.int32).reshape(_NW, _EPW)
    rix = jnp.arange(5 * _ZR, dtype=jnp.int32).reshape(5, _ZR)
    nd = jax.ShapeDtypeStruct((n, d), jnp.float32)

    psrc, pdst = _sc_partition(src2, dst2)
    hist = _sc_degree_hist(dst2, rix, psrc)
    h1 = pl.pallas_call(_mm_body, out_shape=nd)(x, W1)
    deg = (hist[0] + hist[1] + 1.0).reshape(n, 1)
    hs1 = pl.pallas_call(_scale_body, out_shape=nd)(h1, deg)
    agg1 = _sc_edge_agg(hs1, psrc, pdst).reshape(n, d)
    hs2 = pl.pallas_call(
        _mid_body,
        out_shape=nd,
    )(agg1, hs1, deg, b1.reshape(1, d), gamma.reshape(1, d),
      beta.reshape(1, d), W2)
    agg2 = _sc_edge_agg(hs2, psrc, pdst).reshape(n, d)
    out = pl.pallas_call(
        _fin_body,
        out_shape=nd,
    )(agg2, hs2, deg, b2.reshape(1, d))
    return out


# speedup vs baseline: 1.6249x; 1.6249x over previous
import functools

import jax
import jax.numpy as jnp
from jax import lax
from jax.experimental import pallas as pl
from jax.experimental.pallas import tpu as pltpu
from jax.experimental.pallas import tpu_sc as plsc

_NC = 2
_NS = 16
_L = 16
_NW = _NC * _NS

_N = 10000
_D = 128
_E = 320000
_EPW = _E // _NW
_CH = 80
_NCH = _EPW // _CH
_RPS = _N // _NS
_ZR = 125
_HR = _N // _L

_mesh = plsc.VectorSubcoreMesh(core_axis_name="c", subcore_axis_name="s")
_sc_params = pltpu.CompilerParams(needs_layout_passes=False)


@functools.partial(
    pl.kernel,
    out_type=jax.ShapeDtypeStruct((_NC, _HR, _L), jnp.float32),
    mesh=_mesh,
    scratch_types=[
        pltpu.VMEM((_EPW,), jnp.int32),
        pltpu.VMEM((_HR, _L), jnp.float32),
        pltpu.VMEM((5, _ZR), jnp.int32),
        pltpu.VMEM_SHARED((_HR, _L), jnp.float32),
    ],
    compiler_params=_sc_params,
)
def _sc_degree_hist(dst_hbm, rix_hbm, dep_hbm, out_hbm,
                    idx_v, hist_v, rix_v, hist_sh):
    c = lax.axis_index("c")
    s = lax.axis_index("s")
    wid = s * _NC + c
    pltpu.sync_copy(dst_hbm.at[wid], idx_v)
    pltpu.sync_copy(rix_hbm, rix_v)
    z16 = jnp.zeros((_L,), jnp.float32)

    @pl.loop(0, _HR)
    def _zero(r):
        hist_v[r, :] = z16

    @pl.when(s == 0)
    def _():
        pltpu.sync_copy(hist_v, hist_sh)

    plsc.subcore_barrier()

    ones = jnp.ones((_L,), jnp.float32)
    four = jnp.full((_L,), 4, jnp.int32)
    fifteen = jnp.full((_L,), 15, jnp.int32)

    @pl.loop(0, _EPW, step=_L)
    def _hist(i):
        idx = idx_v[pl.ds(i, _L)]
        row = jnp.right_shift(idx, four)
        colv = jnp.bitwise_and(idx, fifteen)
        plsc.addupdate_scatter(hist_v, [row, colv], ones)

    @pl.loop(0, 5)
    def _combine(k):
        pltpu.sync_copy(hist_v.at[pl.ds(k * _ZR, _ZR)],
                        hist_sh.at[rix_v.at[k]], add=True)

    plsc.subcore_barrier()

    @pl.when(s == 0)
    def _():
        pltpu.sync_copy(hist_sh, out_hbm.at[c])


_HALF = _N // _NC
_TRASH = 64
_AROWS = 5120
_CHP = 80
_BCAP = 5440
_BCH = _BCAP // _CHP
_NSEG = 2 * _BCH


@functools.partial(
    pl.kernel,
    out_type=[
        jax.ShapeDtypeStruct((_NW, 2, _BCH, _CHP), jnp.int32),
        jax.ShapeDtypeStruct((_NW, 2, _BCH, _CHP), jnp.int32),
    ],
    mesh=_mesh,
    scratch_types=[
        pltpu.VMEM((_EPW,), jnp.int32),
        pltpu.VMEM((_EPW,), jnp.int32),
        pltpu.VMEM((_BCH, _CHP), jnp.int32),
        pltpu.VMEM((_BCH, _CHP), jnp.int32),
        pltpu.VMEM((_BCH, _CHP), jnp.int32),
        pltpu.VMEM((_BCH, _CHP), jnp.int32),
    ],
    compiler_params=_sc_params,
)
def _sc_partition(src_hbm, dst_hbm, psrc_hbm, pdst_hbm,
                  sv, dv, cs0, cd0, cs1, cd1):
    c = lax.axis_index("c")
    s = lax.axis_index("s")
    wid = s * _NC + c
    pltpu.sync_copy(src_hbm.at[wid], sv)
    pltpu.sync_copy(dst_hbm.at[wid], dv)
    iota = lax.iota(jnp.int32, _L)
    chw = jnp.full((_L,), _CHP, jnp.int32)
    halfv = jnp.full((_L,), _HALF, jnp.int32)
    capv = jnp.full((_L,), _BCAP - 1, jnp.int32)
    zsrc = jnp.zeros((_L,), jnp.int32)
    tmask = jnp.full((_L,), _TRASH - 1, jnp.int32)

    @pl.loop(0, _BCH)
    def _prefill(r):
        @pl.loop(0, _CHP, step=_L)
        def _(j):
            trash = halfv + jnp.bitwise_and(iota + (r * _CHP + j), tmask)
            cs0[r, pl.ds(j, _L)] = zsrc
            cd0[r, pl.ds(j, _L)] = trash
            cs1[r, pl.ds(j, _L)] = zsrc
            cd1[r, pl.ds(j, _L)] = trash

    def vec_body(v, carry):
        b0, b1 = carry
        s16 = sv[pl.ds(v * _L, _L)]
        d16 = dv[pl.ds(v * _L, _L)]
        m0 = d16 < halfv
        m1 = jnp.logical_not(m0)
        cum0 = plsc.cumsum(m0.astype(jnp.int32))
        cum1 = plsc.cumsum(m1.astype(jnp.int32))
        pos0 = jnp.minimum(cum0 + (b0 - 1), capv)
        pos1 = jnp.minimum(cum1 + (b1 - 1), capv)
        r0 = lax.div(pos0, chw)
        c0 = pos0 - r0 * chw
        r1 = lax.div(pos1, chw)
        c1 = pos1 - r1 * chw
        plsc.store_scatter(cs0, [r0, c0], s16, mask=m0)
        plsc.store_scatter(cd0, [r0, c0], d16, mask=m0)
        plsc.store_scatter(cs1, [r1, c1], s16, mask=m1)
        plsc.store_scatter(cd1, [r1, c1], d16 - halfv, mask=m1)
        return (b0 + jnp.sum(m0.astype(jnp.int32)),
                b1 + jnp.sum(m1.astype(jnp.int32)))

    b0, b1 = lax.fori_loop(0, _EPW // _L, vec_body,
                           (jnp.int32(0), jnp.int32(0)))

    @pl.when(b0 + b1 == _EPW)
    def _writeout():
        pltpu.sync_copy(cs0, psrc_hbm.at[wid, 0])
        pltpu.sync_copy(cd0, pdst_hbm.at[wid, 0])
        pltpu.sync_copy(cs1, psrc_hbm.at[wid, 1])
        pltpu.sync_copy(cd1, pdst_hbm.at[wid, 1])


@functools.partial(
    pl.kernel,
    out_type=jax.ShapeDtypeStruct((_NC, _HALF, _D), jnp.float32),
    mesh=_mesh,
    scratch_types=[
        pltpu.VMEM((_BCH, _CHP), jnp.int32),
        pltpu.VMEM((_BCH, _CHP), jnp.int32),
        pltpu.VMEM((_BCH, _CHP), jnp.int32),
        pltpu.VMEM((_BCH, _CHP), jnp.int32),
        pltpu.VMEM((_CHP, _D), jnp.float32),
        pltpu.VMEM((_CHP, _D), jnp.float32),
        pltpu.VMEM_SHARED((_AROWS, _D), jnp.float32),
        pltpu.SemaphoreType.DMA,
        pltpu.SemaphoreType.DMA,
    ],
    compiler_params=_sc_params,
)
def _sc_edge_agg(hs_hbm, psrc_hbm, pdst_hbm, out_hbm,
                 srcA, dstA, srcB, dstB, rows_a, rows_b, agg_sh,
                 gsem_a, gsem_b):
    c = lax.axis_index("c")
    s = lax.axis_index("s")
    z16 = jnp.zeros((_L,), jnp.float32)

    pltpu.sync_copy(psrc_hbm.at[s * 2, c], srcA)
    pltpu.sync_copy(pdst_hbm.at[s * 2, c], dstA)
    pltpu.sync_copy(psrc_hbm.at[s * 2 + 1, c], srcB)
    pltpu.sync_copy(pdst_hbm.at[s * 2 + 1, c], dstB)

    @pl.loop(0, _CHP)
    def _zero(r):
        @pl.loop(0, _D, step=_L)
        def _(j):
            rows_a[r, pl.ds(j, _L)] = z16

    @pl.loop(0, 320, step=80)
    def _zero_spmem(r0):
        pltpu.sync_copy(rows_a, agg_sh.at[pl.ds(s * 320 + r0, 80)])

    plsc.subcore_barrier()

    for sx, dx in ((srcA, dstA), (srcB, dstB)):
        pltpu.async_copy(hs_hbm.at[sx.at[0]], rows_a, gsem_a)

        @pl.loop(0, _BCH, step=2)
        def _edges(j, sx=sx, dx=dx):
            pltpu.async_copy(hs_hbm.at[sx.at[j + 1]], rows_b, gsem_b)
            pltpu.make_async_copy(hs_hbm.at[sx.at[j]], rows_a, gsem_a).wait()
            pltpu.sync_copy(rows_a, agg_sh.at[dx.at[j]], add=True)

            @pl.when(j + 2 < _BCH)
            def _():
                pltpu.async_copy(hs_hbm.at[sx.at[j + 2]], rows_a, gsem_a)

            pltpu.make_async_copy(hs_hbm.at[sx.at[j + 1]], rows_b,
                                  gsem_b).wait()
            pltpu.sync_copy(rows_b, agg_sh.at[dx.at[j + 1]], add=True)

    plsc.subcore_barrier()

    @pl.when(s < 5)
    def _writeout():
        pltpu.sync_copy(agg_sh.at[pl.ds(s * 1000, 1000)],
                        out_hbm.at[c, pl.ds(s * 1000, 1000)])


def _mm_body(x_ref, w_ref, o_ref):
    o_ref[...] = jnp.dot(x_ref[...], w_ref[...],
                         preferred_element_type=jnp.float32)


def _scale_body(h_ref, deg_ref, o_ref):
    o_ref[...] = h_ref[...] * lax.rsqrt(deg_ref[...])


def _mid_body(agg_ref, hs_ref, deg_ref, b1_ref, gamma_ref, beta_ref,
              w2_ref, o_ref):
    dinv = lax.rsqrt(deg_ref[...])
    t = (agg_ref[...] + hs_ref[...]) * dinv + b1_ref[...]
    mu = jnp.mean(t, axis=0, keepdims=True)
    var = jnp.mean(jnp.square(t - mu), axis=0, keepdims=True)
    hn = (t - mu) * lax.rsqrt(var + 1e-5) * gamma_ref[...] + beta_ref[...]
    h = jnp.maximum(hn, 0.0)
    o_ref[...] = jnp.dot(h, w2_ref[...],
                         preferred_element_type=jnp.float32) * dinv


def _fin_body(agg_ref, hs_ref, deg_ref, b2_ref, o_ref):
    o_ref[...] = ((agg_ref[...] + hs_ref[...]) * lax.rsqrt(deg_ref[...])
                  + b2_ref[...])


def kernel(x, edge_index, W1, b1, gamma, beta, W2, b2):
    n, d = x.shape
    src2 = edge_index[0].astype(jnp.int32).reshape(_NW, _EPW)
    dst2 = edge_index[1].astype(jnp.int32).reshape(_NW, _EPW)
    rix = jnp.arange(5 * _ZR, dtype=jnp.int32).reshape(5, _ZR)
    nd = jax.ShapeDtypeStruct((n, d), jnp.float32)

    psrc, pdst = _sc_partition(src2, dst2)
    hist = _sc_degree_hist(dst2, rix, psrc)
    h1 = pl.pallas_call(_mm_body, out_shape=nd)(x, W1)
    deg = (hist[0] + hist[1] + 1.0).reshape(n, 1)
    hs1 = pl.pallas_call(_scale_body, out_shape=nd)(h1, deg)
    agg1 = _sc_edge_agg(hs1, psrc, pdst).reshape(n, d)
    hs2 = pl.pallas_call(
        _mid_body,
        out_shape=nd,
    )(agg1, hs1, deg, b1.reshape(1, d), gamma.reshape(1, d),
      beta.reshape(1, d), W2)
    agg2 = _sc_edge_agg(hs2, psrc, pdst).reshape(n, d)
    out = pl.pallas_call(
        _fin_body,
        out_shape=nd,
    )(agg2, hs2, deg, b2.reshape(1, d))
    return out


# speedup vs baseline: 7.8804x; 4.8499x over previous
import functools

import jax
import jax.numpy as jnp
from jax import lax
from jax.experimental import pallas as pl
from jax.experimental.pallas import tpu as pltpu
from jax.experimental.pallas import tpu_sc as plsc

_NC = 2
_NS = 16
_L = 16
_NW = _NC * _NS

_N = 10000
_D = 128
_E = 320000
_EPW = _E // _NW
_CH = 80
_ZR = 125
_HR = _N // _L

_mesh = plsc.VectorSubcoreMesh(core_axis_name="c", subcore_axis_name="s")
_sc_params = pltpu.CompilerParams(needs_layout_passes=False)


@functools.partial(
    pl.kernel,
    out_type=jax.ShapeDtypeStruct((_NC, _HR, _L), jnp.float32),
    mesh=_mesh,
    scratch_types=[
        pltpu.VMEM((_EPW,), jnp.int32),
        pltpu.VMEM((_HR, _L), jnp.float32),
        pltpu.VMEM((5, _ZR), jnp.int32),
        pltpu.VMEM_SHARED((_HR, _L), jnp.float32),
    ],
    compiler_params=_sc_params,
)
def _sc_degree_hist(dst_hbm, rix_hbm, out_hbm, idx_v, hist_v, rix_v, hist_sh):
    c = lax.axis_index("c")
    s = lax.axis_index("s")
    wid = s * _NC + c
    pltpu.sync_copy(dst_hbm.at[wid], idx_v)
    pltpu.sync_copy(rix_hbm, rix_v)
    z16 = jnp.zeros((_L,), jnp.float32)

    @pl.loop(0, _HR)
    def _zero(r):
        hist_v[r, :] = z16

    @pl.when(s == 0)
    def _():
        pltpu.sync_copy(hist_v, hist_sh)

    plsc.subcore_barrier()

    ones = jnp.ones((_L,), jnp.float32)
    four = jnp.full((_L,), 4, jnp.int32)
    fifteen = jnp.full((_L,), 15, jnp.int32)

    @pl.loop(0, _EPW, step=_L)
    def _hist(i):
        idx = idx_v[pl.ds(i, _L)]
        row = jnp.right_shift(idx, four)
        colv = jnp.bitwise_and(idx, fifteen)
        plsc.addupdate_scatter(hist_v, [row, colv], ones)

    @pl.loop(0, 5)
    def _combine(k):
        pltpu.sync_copy(hist_v.at[pl.ds(k * _ZR, _ZR)],
                        hist_sh.at[rix_v.at[k]], add=True)

    plsc.subcore_barrier()

    @pl.when(s == 0)
    def _():
        pltpu.sync_copy(hist_sh, out_hbm.at[c])


_HALF = _N // _NC
_TRASH = 64
_AROWS = 5120
_EPC = _E // _NS
_NCHC = _EPC // _CH


@functools.partial(
    pl.kernel,
    out_type=jax.ShapeDtypeStruct((_NC, _HALF, _D), jnp.float32),
    mesh=_mesh,
    scratch_types=[
        pltpu.VMEM((_NCHC, _CH), jnp.int32),
        pltpu.VMEM((_NCHC, _CH), jnp.int32),
        pltpu.VMEM((_CH, _D), jnp.float32),
        pltpu.VMEM((_CH, _D), jnp.float32),
        pltpu.VMEM_SHARED((_AROWS, _D), jnp.float32),
        pltpu.SemaphoreType.DMA,
        pltpu.SemaphoreType.DMA,
    ],
    compiler_params=_sc_params,
)
def _sc_edge_agg(hs_hbm, src_hbm, dst_hbm, out_hbm,
                 src_v, dst_v, rows_a, rows_b, agg_sh, gsem_a, gsem_b):
    c = lax.axis_index("c")
    s = lax.axis_index("s")
    pltpu.sync_copy(src_hbm.at[s], src_v)
    pltpu.sync_copy(dst_hbm.at[s], dst_v)
    z16 = jnp.zeros((_L,), jnp.float32)

    @pl.loop(0, _CH)
    def _zero(r):
        @pl.loop(0, _D, step=_L)
        def _(j):
            rows_a[r, pl.ds(j, _L)] = z16

    @pl.loop(0, 320, step=_CH)
    def _zero_spmem(r0):
        pltpu.sync_copy(rows_a, agg_sh.at[pl.ds(s * 320 + r0, _CH)])

    lo = jnp.full((_L,), 0, jnp.int32) + c * _HALF
    hi = lo + _HALF
    tbase = jnp.full((_L,), _HALF, jnp.int32)
    tmask = jnp.full((_L,), _TRASH - 1, jnp.int32)

    @pl.loop(0, _NCHC)
    def _remap(r):
        @pl.loop(0, _CH, step=_L)
        def _(j):
            v = dst_v[r, pl.ds(j, _L)]
            in_half = (v >= lo) & (v < hi)
            mapped = jnp.where(in_half, v - lo,
                               tbase + jnp.bitwise_and(v, tmask))
            dst_v[r, pl.ds(j, _L)] = mapped

    pltpu.async_copy(hs_hbm.at[src_v.at[0]], rows_a, gsem_a)
    plsc.subcore_barrier()

    @pl.loop(0, _NCHC, step=2)
    def _edges(j):
        pltpu.async_copy(hs_hbm.at[src_v.at[j + 1]], rows_b, gsem_b)
        pltpu.make_async_copy(hs_hbm.at[src_v.at[j]], rows_a, gsem_a).wait()
        pltpu.sync_copy(rows_a, agg_sh.at[dst_v.at[j]], add=True)

        @pl.when(j + 2 < _NCHC)
        def _():
            pltpu.async_copy(hs_hbm.at[src_v.at[j + 2]], rows_a, gsem_a)

        pltpu.make_async_copy(hs_hbm.at[src_v.at[j + 1]], rows_b, gsem_b).wait()
        pltpu.sync_copy(rows_b, agg_sh.at[dst_v.at[j + 1]], add=True)

    plsc.subcore_barrier()

    @pl.when(s < 5)
    def _writeout():
        pltpu.sync_copy(agg_sh.at[pl.ds(s * 1000, 1000)],
                        out_hbm.at[c, pl.ds(s * 1000, 1000)])


def _mm_body(x_ref, w_ref, o_ref):
    o_ref[...] = jnp.dot(x_ref[...], w_ref[...],
                         preferred_element_type=jnp.float32)


def _scale_body(h_ref, deg_ref, o_ref):
    o_ref[...] = h_ref[...] * lax.rsqrt(deg_ref[...])


def _mid_body(agg_ref, hs_ref, deg_ref, b1_ref, gamma_ref, beta_ref,
              w2_ref, o_ref):
    dinv = lax.rsqrt(deg_ref[...])
    t = (agg_ref[...] + hs_ref[...]) * dinv + b1_ref[...]
    mu = jnp.mean(t, axis=0, keepdims=True)
    var = jnp.mean(jnp.square(t - mu), axis=0, keepdims=True)
    hn = (t - mu) * lax.rsqrt(var + 1e-5) * gamma_ref[...] + beta_ref[...]
    h = jnp.maximum(hn, 0.0)
    o_ref[...] = jnp.dot(h, w2_ref[...],
                         preferred_element_type=jnp.float32) * dinv


def _fin_body(agg_ref, hs_ref, deg_ref, b2_ref, o_ref):
    o_ref[...] = ((agg_ref[...] + hs_ref[...]) * lax.rsqrt(deg_ref[...])
                  + b2_ref[...])


def kernel(x, edge_index, W1, b1, gamma, beta, W2, b2):
    n, d = x.shape
    src3 = edge_index[0].astype(jnp.int32).reshape(_NS, _NCHC, _CH)
    dst = edge_index[1].astype(jnp.int32)
    dst3 = dst.reshape(_NS, _NCHC, _CH)
    dst2 = dst.reshape(_NW, _EPW)
    rix = jnp.arange(5 * _ZR, dtype=jnp.int32).reshape(5, _ZR)
    nd = jax.ShapeDtypeStruct((n, d), jnp.float32)

    hist = _sc_degree_hist(dst2, rix)
    h1 = pl.pallas_call(_mm_body, out_shape=nd)(x, W1)
    deg = (hist[0] + hist[1] + 1.0).reshape(n, 1)
    hs1 = pl.pallas_call(_scale_body, out_shape=nd)(h1, deg)
    agg1 = _sc_edge_agg(hs1, src3, dst3).reshape(n, d)
    hs2 = pl.pallas_call(
        _mid_body,
        out_shape=nd,
    )(agg1, hs1, deg, b1.reshape(1, d), gamma.reshape(1, d),
      beta.reshape(1, d), W2)
    agg2 = _sc_edge_agg(hs2, src3, dst3).reshape(n, d)
    out = pl.pallas_call(
        _fin_body,
        out_shape=nd,
    )(agg2, hs2, deg, b2.reshape(1, d))
    return out
